# Initial kernel scaffold; baseline (speedup 1.0000x reference)
import jax
import jax.numpy as jnp
from jax.experimental import pallas as pl


def kernel(A, W_enc, W_dec, b_pre):
    raise NotImplementedError("write your pallas kernel here")


# speedup vs baseline: 3.4160x; 3.4160x over previous
import functools

import jax
import jax.numpy as jnp
from jax.experimental import pallas as pl
from jax.experimental.pallas import tpu as pltpu

K = 32


def _encode_topk_kernel(n_j, jt, A_ref, W_ref, b_ref, acts_ref, z_ref):
    j = pl.program_id(1)
    x = A_ref[...] - b_ref[...]
    blk = jnp.dot(x, W_ref[...], preferred_element_type=jnp.float32)
    acts_ref[:, pl.ds(j * jt, jt)] = blk

    @pl.when(j == n_j - 1)
    def _():
        z_ref[...] = acts_ref[...]

        def body(_, m_prev):
            cur = z_ref[...]
            m = jnp.max(cur, axis=1, keepdims=True)
            z_ref[...] = jnp.where(cur >= m, -jnp.inf, cur)
            return m

        m = jax.lax.fori_loop(
            0, K, body, jnp.zeros((z_ref.shape[0], 1), jnp.float32))
        buf = acts_ref[...]
        z_ref[...] = jnp.where(buf >= m, buf, 0.0)


def _decode_kernel(b_ref, z_ref, W_ref, out_ref):
    t = pl.program_id(1)

    @pl.when(t == 0)
    def _():
        out_ref[...] = jnp.broadcast_to(b_ref[...], out_ref.shape)

    out_ref[...] += jnp.dot(z_ref[...], W_ref[...],
                            preferred_element_type=jnp.float32)


@jax.jit
def kernel(A, W_enc, W_dec, b_pre):
    n, d_act = A.shape
    d_hid = W_enc.shape[1]

    R = 128
    JT = 1024
    n_r = n // R
    n_j = d_hid // JT

    b2 = b_pre.reshape(1, d_act)

    acts, z = pl.pallas_call(
        functools.partial(_encode_topk_kernel, n_j, JT),
        grid=(n_r, n_j),
        in_specs=[
            pl.BlockSpec((R, d_act), lambda r, j: (r, 0)),
            pl.BlockSpec((d_act, JT), lambda r, j: (0, j)),
            pl.BlockSpec((1, d_act), lambda r, j: (0, 0)),
        ],
        out_specs=[
            pl.BlockSpec((R, d_hid), lambda r, j: (r, 0)),
            pl.BlockSpec((R, d_hid), lambda r, j: (r, 0)),
        ],
        out_shape=[
            jax.ShapeDtypeStruct((n, d_hid), jnp.float32),
            jax.ShapeDtypeStruct((n, d_hid), jnp.float32),
        ],
        compiler_params=pltpu.CompilerParams(
            dimension_semantics=("arbitrary", "arbitrary"),
            vmem_limit_bytes=100 * 1024 * 1024,
        ),
    )(A, W_enc, b2)

    RT = 256
    KT = 2048
    n_t = d_hid // KT

    out = pl.pallas_call(
        functools.partial(_decode_kernel),
        grid=(n // RT, n_t),
        in_specs=[
            pl.BlockSpec((1, d_act), lambda r, t: (0, 0)),
            pl.BlockSpec((RT, KT), lambda r, t: (r, t)),
            pl.BlockSpec((KT, d_act), lambda r, t: (t, 0)),
        ],
        out_specs=pl.BlockSpec((RT, d_act), lambda r, t: (r, 0)),
        out_shape=jax.ShapeDtypeStruct((n, d_act), jnp.float32),
        compiler_params=pltpu.CompilerParams(
            dimension_semantics=("arbitrary", "arbitrary"),
        ),
    )(b2, z, W_dec)

    return (out, acts, z)


# speedup vs baseline: 4.3792x; 1.2819x over previous
import functools

import jax
import jax.numpy as jnp
from jax.experimental import pallas as pl
from jax.experimental.pallas import tpu as pltpu

K = 32


def _encode_topk_kernel(n_j, jt, A_ref, W_ref, b_ref, acts_ref, z_ref, x_ref):
    j = pl.program_id(1)

    @pl.when(j == 0)
    def _():
        x_ref[...] = A_ref[...] - b_ref[...]

    blk = jnp.dot(x_ref[...], W_ref[...], preferred_element_type=jnp.float32)
    acts_ref[:, pl.ds(j * jt, jt)] = blk

    @pl.when(j == n_j - 1)
    def _():
        buf = acts_ref[...]
        r = buf.shape[0]

        s = buf
        for _ in range(4):
            h = s.shape[1] // 2
            s = jnp.maximum(s[:, :h], s[:, h:])

        def body(_, carry):
            cur, _ = carry
            m = jnp.max(cur, axis=1, keepdims=True)
            cur = jnp.where(cur >= m, -jnp.inf, cur)
            return cur, m

        _, t = jax.lax.fori_loop(
            0, K, body, (s, jnp.zeros((r, 1), jnp.float32)))

        c = jnp.sum((buf >= t).astype(jnp.float32), axis=1, keepdims=True)

        def w_cond(st):
            return jnp.max(st[1]) > float(K)

        def w_body(st):
            t, c = st
            need = c > float(K)
            tn = jnp.min(jnp.where(buf > t, buf, jnp.inf), axis=1,
                         keepdims=True)
            return (jnp.where(need, tn, t), jnp.where(need, c - 1.0, c))

        t, c = jax.lax.while_loop(w_cond, w_body, (t, c))
        z_ref[...] = jnp.where(buf >= t, buf, 0.0)


def _decode_kernel(b_ref, z_ref, W_ref, out_ref):
    t = pl.program_id(1)

    @pl.when(t == 0)
    def _():
        out_ref[...] = jnp.broadcast_to(b_ref[...], out_ref.shape)

    out_ref[...] += jnp.dot(z_ref[...], W_ref[...],
                            preferred_element_type=jnp.float32)


@jax.jit
def kernel(A, W_enc, W_dec, b_pre):
    n, d_act = A.shape
    d_hid = W_enc.shape[1]

    R = 128
    JT = 1024
    n_r = n // R
    n_j = d_hid // JT

    b2 = b_pre.reshape(1, d_act)

    acts, z = pl.pallas_call(
        functools.partial(_encode_topk_kernel, n_j, JT),
        grid=(n_r, n_j),
        in_specs=[
            pl.BlockSpec((R, d_act), lambda r, j: (r, 0)),
            pl.BlockSpec((d_act, JT), lambda r, j: (0, j)),
            pl.BlockSpec((1, d_act), lambda r, j: (0, 0)),
        ],
        out_specs=[
            pl.BlockSpec((R, d_hid), lambda r, j: (r, 0)),
            pl.BlockSpec((R, d_hid), lambda r, j: (r, 0)),
        ],
        out_shape=[
            jax.ShapeDtypeStruct((n, d_hid), jnp.float32),
            jax.ShapeDtypeStruct((n, d_hid), jnp.float32),
        ],
        scratch_shapes=[pltpu.VMEM((R, d_act), jnp.float32)],
        compiler_params=pltpu.CompilerParams(
            dimension_semantics=("arbitrary", "arbitrary"),
            vmem_limit_bytes=100 * 1024 * 1024,
        ),
    )(A, W_enc, b2)

    RT = 256
    KT = 2048
    n_t = d_hid // KT

    out = pl.pallas_call(
        functools.partial(_decode_kernel),
        grid=(n // RT, n_t),
        in_specs=[
            pl.BlockSpec((1, d_act), lambda r, t: (0, 0)),
            pl.BlockSpec((RT, KT), lambda r, t: (r, t)),
            pl.BlockSpec((KT, d_act), lambda r, t: (t, 0)),
        ],
        out_specs=pl.BlockSpec((RT, d_act), lambda r, t: (r, 0)),
        out_shape=jax.ShapeDtypeStruct((n, d_act), jnp.float32),
        compiler_params=pltpu.CompilerParams(
            dimension_semantics=("arbitrary", "arbitrary"),
        ),
    )(b2, z, W_dec)

    return (out, acts, z)


# speedup vs baseline: 6.6748x; 1.5242x over previous
import functools

import jax
import jax.numpy as jnp
from jax.experimental import pallas as pl
from jax.experimental.pallas import tpu as pltpu

K = 32


def _fold_max(x, n):
    for _ in range(n):
        h = x.shape[1] // 2
        x = jnp.maximum(x[:, :h], x[:, h:])
    return x


def _raise_to_k(x, t, c):
    def w_cond(st):
        return jnp.max(st[1]) > float(K)

    def w_body(st):
        t, c = st
        need = c > float(K)
        tn = jnp.min(jnp.where(x > t, x, jnp.inf), axis=1, keepdims=True)
        return (jnp.where(need, tn, t), jnp.where(need, c - 1.0, c))

    t, c = jax.lax.while_loop(w_cond, w_body, (t, c))
    return t


def _encode_topk_kernel(n_j, jt, A_ref, W_ref, acts_ref, z_ref, s_ref):
    j = pl.program_id(1)
    blk = jnp.dot(A_ref[...], W_ref[...], preferred_element_type=jnp.float32)
    acts_ref[:, pl.ds(j * jt, jt)] = blk
    gt = jt // 8
    s_ref[:, pl.ds(j * gt, gt)] = _fold_max(blk, 3)

    @pl.when(j == n_j - 1)
    def _():
        s = s_ref[...]
        r = s.shape[0]
        s2 = _fold_max(s, 4)

        def body(_, carry):
            cur, _ = carry
            m = jnp.max(cur, axis=1, keepdims=True)
            cur = jnp.where(cur >= m, -jnp.inf, cur)
            return cur, m

        _, t2 = jax.lax.fori_loop(0, K, body,
                                  (s2, jnp.zeros((r, 1), jnp.float32)))

        c_s = jnp.sum((s >= t2).astype(jnp.float32), axis=1, keepdims=True)
        t_s = _raise_to_k(s, t2, c_s)

        acts = acts_ref[...]
        c = jnp.sum((acts >= t_s).astype(jnp.float32), axis=1, keepdims=True)
        t = _raise_to_k(acts, t_s, c)

        z_ref[...] = jnp.where(acts >= t, acts, 0.0)


def _decode_kernel(b_ref, z_ref, W_ref, out_ref):
    t = pl.program_id(1)

    @pl.when(t == 0)
    def _():
        out_ref[...] = jnp.broadcast_to(b_ref[...], out_ref.shape)

    out_ref[...] += jnp.dot(z_ref[...].astype(jnp.bfloat16), W_ref[...],
                            preferred_element_type=jnp.float32)


@jax.jit
def kernel(A, W_enc, W_dec, b_pre):
    n, d_act = A.shape
    d_hid = W_enc.shape[1]

    R = 128
    JT = 1024
    n_r = n // R
    n_j = d_hid // JT

    x = (A - b_pre).astype(jnp.bfloat16)
    W_bf = W_enc.astype(jnp.bfloat16)
    Wd_bf = W_dec.astype(jnp.bfloat16)
    b2 = b_pre.reshape(1, d_act)

    acts, z = pl.pallas_call(
        functools.partial(_encode_topk_kernel, n_j, JT),
        grid=(n_r, n_j),
        in_specs=[
            pl.BlockSpec((R, d_act), lambda r, j: (r, 0)),
            pl.BlockSpec((d_act, JT), lambda r, j: (0, j)),
        ],
        out_specs=[
            pl.BlockSpec((R, d_hid), lambda r, j: (r, 0)),
            pl.BlockSpec((R, d_hid), lambda r, j: (r, 0)),
        ],
        out_shape=[
            jax.ShapeDtypeStruct((n, d_hid), jnp.float32),
            jax.ShapeDtypeStruct((n, d_hid), jnp.float32),
        ],
        scratch_shapes=[pltpu.VMEM((R, d_hid // 8), jnp.float32)],
        compiler_params=pltpu.CompilerParams(
            dimension_semantics=("arbitrary", "arbitrary"),
            vmem_limit_bytes=100 * 1024 * 1024,
        ),
    )(x, W_bf)

    RT = min(1024, n)
    KT = min(2048, d_hid)
    n_t = d_hid // KT

    out = pl.pallas_call(
        _decode_kernel,
        grid=(n // RT, n_t),
        in_specs=[
            pl.BlockSpec((1, d_act), lambda r, t: (0, 0)),
            pl.BlockSpec((RT, KT), lambda r, t: (r, t)),
            pl.BlockSpec((KT, d_act), lambda r, t: (t, 0)),
        ],
        out_specs=pl.BlockSpec((RT, d_act), lambda r, t: (r, 0)),
        out_shape=jax.ShapeDtypeStruct((n, d_act), jnp.float32),
        compiler_params=pltpu.CompilerParams(
            dimension_semantics=("arbitrary", "arbitrary"),
            vmem_limit_bytes=100 * 1024 * 1024,
        ),
    )(b2, z, Wd_bf)

    return (out, acts, z)


# speedup vs baseline: 9.0159x; 1.3507x over previous
import functools

import jax
import jax.numpy as jnp
from jax.experimental import pallas as pl
from jax.experimental.pallas import tpu as pltpu

K = 32


def _fold_max(x, n):
    for _ in range(n):
        h = x.shape[1] // 2
        x = jnp.maximum(x[:, :h], x[:, h:])
    return x


def _raise_to_k(x, t, c):
    def w_cond(st):
        return jnp.max(st[1]) > float(K)

    def w_body(st):
        t, c = st
        need = c > float(K)
        tn = jnp.min(jnp.where(x > t, x, jnp.inf), axis=1, keepdims=True)
        return (jnp.where(need, tn, t), jnp.where(need, c - 1.0, c))

    t, c = jax.lax.while_loop(w_cond, w_body, (t, c))
    return t


def _select_topk_threshold(ab, sb):
    s = sb[...]
    r = s.shape[0]
    s2 = _fold_max(s, 4)

    def body(_, carry):
        cur, _ = carry
        m = jnp.max(cur, axis=1, keepdims=True)
        cur = jnp.where(cur >= m, -jnp.inf, cur)
        return cur, m

    _, t2 = jax.lax.fori_loop(0, K, body,
                              (s2, jnp.zeros((r, 1), jnp.float32)))

    c_s = jnp.sum((s >= t2).astype(jnp.float32), axis=1, keepdims=True)
    t_s = _raise_to_k(s, t2, c_s)

    acts = ab[...]
    c = jnp.sum((acts >= t_s).astype(jnp.float32), axis=1, keepdims=True)
    return _raise_to_k(acts, t_s, c)


def _encode_topk_kernel(n_r, n_j, rb, jt, A_ref, W_ref, acts_ref, thr_ref,
                        ab, sb):
    r = pl.program_id(0)
    j = pl.program_id(1)
    gt = jt // 8

    @pl.when(jnp.logical_and(r >= 1, j == 0))
    def _():
        thr_ref[pl.ds((r - 1) * rb, rb), :] = _select_topk_threshold(ab, sb)

    @pl.when(r < n_r)
    def _():
        blk = jnp.dot(A_ref[...], W_ref[...],
                      preferred_element_type=jnp.float32)
        acts_ref[...] = blk
        ab[:, pl.ds(j * jt, jt)] = blk
        sb[:, pl.ds(j * gt, gt)] = _fold_max(blk, 3)


def _zdecode_kernel(b_ref, acts_ref, thr_ref, W_ref, out_ref, z_ref, *, rt):
    r = pl.program_id(0)
    t = pl.program_id(1)

    a = acts_ref[...]
    thr = thr_ref[pl.ds(r * rt, rt), :]
    zt = jnp.where(a >= thr, a, 0.0)
    z_ref[...] = zt

    @pl.when(t == 0)
    def _():
        out_ref[...] = jnp.broadcast_to(b_ref[...], out_ref.shape)

    out_ref[...] += jnp.dot(zt.astype(jnp.bfloat16), W_ref[...],
                            preferred_element_type=jnp.float32)


@jax.jit
def kernel(A, W_enc, W_dec, b_pre):
    n, d_act = A.shape
    d_hid = W_enc.shape[1]

    R = 256
    JT = 1024
    n_r = n // R
    n_j = d_hid // JT
    last = n_r - 1

    x = (A - b_pre).astype(jnp.bfloat16)
    W_bf = W_enc.astype(jnp.bfloat16)
    Wd_bf = W_dec.astype(jnp.bfloat16)
    b2 = b_pre.reshape(1, d_act)

    acts, thr = pl.pallas_call(
        functools.partial(_encode_topk_kernel, n_r, n_j, R, JT),
        grid=(n_r + 1, n_j),
        in_specs=[
            pl.BlockSpec((R, d_act), lambda r, j: (jnp.minimum(r, last), 0)),
            pl.BlockSpec((d_act, JT), lambda r, j: (0, j)),
        ],
        out_specs=[
            pl.BlockSpec(
                (R, JT),
                lambda r, j: (jnp.minimum(r, last),
                              jnp.where(r < n_r, j, n_j - 1))),
            pl.BlockSpec((n, 1), lambda r, j: (0, 0)),
        ],
        out_shape=[
            jax.ShapeDtypeStruct((n, d_hid), jnp.float32),
            jax.ShapeDtypeStruct((n, 1), jnp.float32),
        ],
        scratch_shapes=[
            pltpu.VMEM((R, d_hid), jnp.float32),
            pltpu.VMEM((R, d_hid // 8), jnp.float32),
        ],
        compiler_params=pltpu.CompilerParams(
            dimension_semantics=("arbitrary", "arbitrary"),
            vmem_limit_bytes=100 * 1024 * 1024,
        ),
    )(x, W_bf)

    RT = min(1024, n)
    KT = min(1024, d_hid)
    n_t = d_hid // KT

    out, z = pl.pallas_call(
        functools.partial(_zdecode_kernel, rt=RT),
        grid=(n // RT, n_t),
        in_specs=[
            pl.BlockSpec((1, d_act), lambda r, t: (0, 0)),
            pl.BlockSpec((RT, KT), lambda r, t: (r, t)),
            pl.BlockSpec((n, 1), lambda r, t: (0, 0)),
            pl.BlockSpec((KT, d_act), lambda r, t: (t, 0)),
        ],
        out_specs=[
            pl.BlockSpec((RT, d_act), lambda r, t: (r, 0)),
            pl.BlockSpec((RT, KT), lambda r, t: (r, t)),
        ],
        out_shape=[
            jax.ShapeDtypeStruct((n, d_act), jnp.float32),
            jax.ShapeDtypeStruct((n, d_hid), jnp.float32),
        ],
        compiler_params=pltpu.CompilerParams(
            dimension_semantics=("arbitrary", "arbitrary"),
            vmem_limit_bytes=100 * 1024 * 1024,
        ),
    )(b2, acts, thr, Wd_bf)

    return (out, acts, z)
